# Initial kernel scaffold; baseline (speedup 1.0000x reference)
import functools

import jax
import jax.numpy as jnp
from jax import lax
from jax.experimental import pallas as pl
from jax.experimental.pallas import tpu as pltpu
from jax.experimental.pallas import tpu_sc as plsc

F32 = jnp.float32
_COEFF = -0.5
_OFFSET = (0., 1., 1.25, 1.5, 1.75, 2., 2.25, 2.5, 2.75, 3., 3.5, 4., 4.5,
           5., 5.5, 6., 7., 8., 9., 10.)
_W = 144
_NC = 2
_NS = 16
_NW = _NC * _NS
_NPAD = 10400
_KG = 400
_RB = 400
_EB = 512


def _silu(v):
    return v * jax.nn.sigmoid(v)


def _sc_gather(table_a, table_b, dst, src):
    e = dst.shape[0]
    per_w = e // _NW
    chunks = per_w // _KG
    mesh = plsc.VectorSubcoreMesh(core_axis_name="c", subcore_axis_name="s")

    @functools.partial(
        pl.kernel, mesh=mesh,
        out_type=[jax.ShapeDtypeStruct((e, _W), F32),
                  jax.ShapeDtypeStruct((e, _W), F32)],
        scratch_types=[
            pltpu.VMEM((_KG,), jnp.int32),
            pltpu.VMEM((_KG,), jnp.int32),
            pltpu.VMEM((_KG, _W), F32),
            pltpu.VMEM((_KG, _W), F32),
            pltpu.SemaphoreType.DMA,
            pltpu.SemaphoreType.DMA,
        ])
    def gk(ta, tb, d_h, s_h, out_a, out_b, ia, ib, ba, bb, sa, sb):
        wid = lax.axis_index("s") * _NC + lax.axis_index("c")
        base = wid * per_w

        def body(k, carry):
            off = base + k * _KG
            pltpu.sync_copy(d_h.at[pl.ds(off, _KG)], ia)
            pltpu.sync_copy(s_h.at[pl.ds(off, _KG)], ib)
            ca = pltpu.async_copy(ta.at[ia], ba, sa)
            cb = pltpu.async_copy(tb.at[ib], bb, sb)
            ca.wait()
            cb.wait()
            pltpu.sync_copy(ba, out_a.at[pl.ds(off, _KG)])
            pltpu.sync_copy(bb, out_b.at[pl.ds(off, _KG)])
            return carry

        lax.fori_loop(0, chunks, body, 0)

    return gk(table_a, table_b, dst, src)


def _sc_scatter(m, dst, zeros_hbm):
    e = m.shape[0]
    per_w = e // _NW
    chunks = per_w // _KG
    rows_t = _NPAD // _NS
    mesh = plsc.VectorSubcoreMesh(core_axis_name="c", subcore_axis_name="s")

    @functools.partial(
        pl.kernel, mesh=mesh,
        out_type=jax.ShapeDtypeStruct((2 * _NPAD, _W), F32),
        scratch_types=[
            pltpu.VMEM((_KG,), jnp.int32),
            pltpu.VMEM((_KG, _W), F32),
            pltpu.VMEM_SHARED((_NPAD, _W), F32),
        ])
    def sk(m_h, d_h, z_h, out, idx, buf, acc):
        c = lax.axis_index("c")
        s = lax.axis_index("s")
        wid = s * _NC + c
        r0 = s * rows_t
        pltpu.sync_copy(z_h.at[pl.ds(r0, rows_t)], acc.at[pl.ds(r0, rows_t)])
        plsc.subcore_barrier()
        base = wid * per_w

        def body(k, carry):
            off = base + k * _KG
            pltpu.sync_copy(d_h.at[pl.ds(off, _KG)], idx)
            pltpu.sync_copy(m_h.at[pl.ds(off, _KG)], buf)
            pltpu.sync_copy(buf, acc.at[idx], add=True)
            return carry

        lax.fori_loop(0, chunks, body, 0)
        plsc.subcore_barrier()
        pltpu.sync_copy(acc.at[pl.ds(r0, rows_t)],
                        out.at[pl.ds(c * _NPAD + r0, rows_t)])

    return sk(m, dst, zeros_hbm)


def _prep_tables(h, x8, w0a, w0b, b0):
    n = h.shape[0]

    def body(h_ref, x_ref, wa_ref, wb_ref, b0_ref, a_ref, b_ref):
        hv = h_ref[...]
        ha = jnp.dot(hv, wa_ref[...], preferred_element_type=F32) + b0_ref[...]
        hb = jnp.dot(hv, wb_ref[...], preferred_element_type=F32)
        xv = x_ref[...]
        pad = jnp.zeros((_RB, _W - 136), F32)
        a_ref[...] = jnp.concatenate([ha, xv, pad], axis=1)
        b_ref[...] = jnp.concatenate([hb, -xv, pad], axis=1)

    return pl.pallas_call(
        body,
        grid=(n // _RB,),
        in_specs=[
            pl.BlockSpec((_RB, 128), lambda i: (i, 0)),
            pl.BlockSpec((_RB, 8), lambda i: (i, 0)),
            pl.BlockSpec((128, 128), lambda i: (0, 0)),
            pl.BlockSpec((128, 128), lambda i: (0, 0)),
            pl.BlockSpec((1, 128), lambda i: (0, 0)),
        ],
        out_specs=[pl.BlockSpec((_RB, _W), lambda i: (i, 0)),
                   pl.BlockSpec((_RB, _W), lambda i: (i, 0))],
        out_shape=[jax.ShapeDtypeStruct((n, _W), F32)] * 2,
    )(h, x8, w0a, w0b, b0)


def _edge_mlp(g_a, g_b, w0c, w1, b1, infw, infb, xw0, xb0, xw1r, offs):
    e = g_a.shape[0]

    def body(ga_ref, gb_ref, w0c_ref, w1_ref, b1_ref, iw_ref, ib_ref,
             xw0_ref, xb0_ref, xw1_ref, off_ref, o_ref):
        g = ga_ref[...] + gb_ref[...]
        gsum = g[:, :128]
        relx = g[:, 128:136]
        dsq = jnp.sum(relx * relx, axis=1, keepdims=True)
        dist = jnp.sqrt(dsq + 1e-8)
        dfeat = jnp.exp(_COEFF * (dist - off_ref[...]) ** 2)
        t = gsum + jnp.dot(dfeat, w0c_ref[...], preferred_element_type=F32)
        mij = _silu(t)
        m2 = _silu(jnp.dot(mij, w1_ref[...], preferred_element_type=F32)
                   + b1_ref[...])
        ei = jax.nn.sigmoid(jnp.sum(m2 * iw_ref[...], axis=1, keepdims=True)
                            + ib_ref[0, 0])
        om = m2 * ei
        h1 = _silu(jnp.dot(m2, xw0_ref[...], preferred_element_type=F32)
                   + xb0_ref[...])
        xw = jnp.tanh(jnp.sum(h1 * xw1_ref[...], axis=1, keepdims=True))
        ox = relx * (xw / (dist + 1.0))
        pad = jnp.zeros((_EB, _W - 136), F32)
        o_ref[...] = jnp.concatenate([om, ox, pad], axis=1)

    wfull = lambda shape: pl.BlockSpec(shape, lambda i: (0, 0))
    return pl.pallas_call(
        body,
        grid=(e // _EB,),
        in_specs=[
            pl.BlockSpec((_EB, _W), lambda i: (i, 0)),
            pl.BlockSpec((_EB, _W), lambda i: (i, 0)),
            wfull((20, 128)),
            wfull((128, 128)),
            wfull((1, 128)),
            wfull((1, 128)),
            wfull((1, 128)),
            wfull((128, 128)),
            wfull((1, 128)),
            wfull((1, 128)),
            wfull((1, 20)),
        ],
        out_specs=pl.BlockSpec((_EB, _W), lambda i: (i, 0)),
        out_shape=jax.ShapeDtypeStruct((e, _W), F32),
    )(g_a, g_b, w0c, w1, b1, infw, infb, xw0, xb0, xw1r, offs)


def _node_update(p, h, x8, nw0a, nw0b, nb0, nw1, nb1, nxt):
    n = h.shape[0]
    has_next = nxt is not None
    p1_off = _NPAD // _RB

    def body(*refs):
        if has_next:
            (p0_ref, p1_ref, h_ref, x_ref, wa_ref, wb_ref, b0_ref, w1_ref,
             b1_ref, na_ref, nb_ref, nbias_ref, h_out, x_out, a_out, b_out) = refs
        else:
            (p0_ref, p1_ref, h_ref, x_ref, wa_ref, wb_ref, b0_ref, w1_ref,
             b1_ref, h_out, x_out) = refs
        pv = p0_ref[...] + p1_ref[...]
        mi = pv[:, :128]
        dx = pv[:, 128:136]
        hv = h_ref[...]
        nh = _silu(jnp.dot(mi, wa_ref[...], preferred_element_type=F32)
                   + jnp.dot(hv, wb_ref[...], preferred_element_type=F32)
                   + b0_ref[...])
        hn = hv + jnp.dot(nh, w1_ref[...], preferred_element_type=F32) + b1_ref[...]
        xn = x_ref[...] + dx
        h_out[...] = hn
        x_out[...] = xn
        if has_next:
            ha = jnp.dot(hn, na_ref[...], preferred_element_type=F32) + nbias_ref[...]
            hb = jnp.dot(hn, nb_ref[...], preferred_element_type=F32)
            pad = jnp.zeros((_RB, _W - 136), F32)
            a_out[...] = jnp.concatenate([ha, xn, pad], axis=1)
            b_out[...] = jnp.concatenate([hb, -xn, pad], axis=1)

    wfull = lambda shape: pl.BlockSpec(shape, lambda i: (0, 0))
    in_specs = [
        pl.BlockSpec((_RB, _W), lambda i: (i, 0)),
        pl.BlockSpec((_RB, _W), lambda i: (i + p1_off, 0)),
        pl.BlockSpec((_RB, 128), lambda i: (i, 0)),
        pl.BlockSpec((_RB, 8), lambda i: (i, 0)),
        wfull((128, 128)),
        wfull((128, 128)),
        wfull((1, 128)),
        wfull((128, 128)),
        wfull((1, 128)),
    ]
    out_specs = [pl.BlockSpec((_RB, 128), lambda i: (i, 0)),
                 pl.BlockSpec((_RB, 8), lambda i: (i, 0))]
    out_shape = [jax.ShapeDtypeStruct((n, 128), F32),
                 jax.ShapeDtypeStruct((n, 8), F32)]
    args = [p, p, h, x8, nw0a, nw0b, nb0, nw1, nb1]
    if has_next:
        in_specs += [wfull((128, 128)), wfull((128, 128)), wfull((1, 128))]
        out_specs += [pl.BlockSpec((_RB, _W), lambda i: (i, 0)),
                      pl.BlockSpec((_RB, _W), lambda i: (i, 0))]
        out_shape += [jax.ShapeDtypeStruct((n, _W), F32)] * 2
        args += list(nxt)
    return pl.pallas_call(
        body,
        grid=(n // _RB,),
        in_specs=in_specs,
        out_specs=out_specs,
        out_shape=out_shape,
    )(*args)


def kernel(h, x, edge_index, mask_ligand, edge_w0, edge_b0, edge_w1, edge_b1,
           inf_w, inf_b, x_w0, x_b0, x_w1, node_w0, node_b0, node_w1, node_b1):
    del mask_ligand
    src = edge_index[0].astype(jnp.int32)
    dst = edge_index[1].astype(jnp.int32)
    n = h.shape[0]
    x8 = jnp.pad(x.astype(F32), ((0, 0), (0, 5)))
    zeros_hbm = jnp.zeros((_NPAD, _W), F32)
    offs = jnp.asarray(_OFFSET, F32).reshape(1, 20)

    tab = lambda l: (edge_w0[l, :128], edge_w0[l, 128:256],
                     edge_b0[l].reshape(1, 128))
    a_t, b_t = _prep_tables(h, x8, *tab(0))
    for l in range(2):
        g_a, g_b = _sc_gather(a_t, b_t, dst, src)
        m = _edge_mlp(
            g_a, g_b, edge_w0[l, 256:276], edge_w1[l], edge_b1[l].reshape(1, 128),
            inf_w[l].reshape(1, 128),
            jnp.broadcast_to(inf_b[l].reshape(1, 1), (1, 128)),
            x_w0[l], x_b0[l].reshape(1, 128), x_w1[l].reshape(1, 128), offs)
        p = _sc_scatter(m, dst, zeros_hbm)
        nxt = tab(1) if l == 0 else None
        res = _node_update(p, h, x8, node_w0[l, :128], node_w0[l, 128:256],
                           node_b0[l].reshape(1, 128), node_w1[l],
                           node_b1[l].reshape(1, 128), nxt)
        if l == 0:
            h, x8, a_t, b_t = res
        else:
            h, x8 = res
    return h, x8[:, :3]


# speedup vs baseline: 3.0995x; 3.0995x over previous
import functools

import jax
import jax.numpy as jnp
from jax import lax
from jax.experimental import pallas as pl
from jax.experimental.pallas import tpu as pltpu
from jax.experimental.pallas import tpu_sc as plsc

F32 = jnp.float32
_COEFF = -0.5
_OFFSET = (0., 1., 1.25, 1.5, 1.75, 2., 2.25, 2.5, 2.75, 3., 3.5, 4., 4.5,
           5., 5.5, 6., 7., 8., 9., 10.)
_H = 128
_NC = 2
_NS = 16
_NW = _NC * _NS
_NPAD = 12800
_KG = 400
_KX = 2000
_RB = 400
_EB = 512

_SC_PARAMS = pltpu.CompilerParams(needs_layout_passes=False)


def _silu(v):
    return v * jax.nn.sigmoid(v)


def _sc_gather(table_a, table_b, dst, src):
    e = dst.shape[0]
    per_w = e // _NW
    chunks = per_w // _KG
    mesh = plsc.VectorSubcoreMesh(core_axis_name="c", subcore_axis_name="s")

    @functools.partial(
        pl.kernel, mesh=mesh, compiler_params=_SC_PARAMS,
        out_type=[jax.ShapeDtypeStruct((e, _H), F32),
                  jax.ShapeDtypeStruct((e, _H), F32)],
        scratch_types=[
            pltpu.VMEM((_KG,), jnp.int32),
            pltpu.VMEM((_KG,), jnp.int32),
            pltpu.VMEM((_KG, _H), F32),
            pltpu.VMEM((_KG, _H), F32),
            pltpu.SemaphoreType.DMA,
            pltpu.SemaphoreType.DMA,
        ])
    def gk(ta, tb, d_h, s_h, out_a, out_b, ia, ib, ba, bb, sa, sb):
        wid = lax.axis_index("s") * _NC + lax.axis_index("c")
        base = wid * per_w

        def body(k, carry):
            off = base + k * _KG
            pltpu.sync_copy(d_h.at[pl.ds(off, _KG)], ia)
            pltpu.sync_copy(s_h.at[pl.ds(off, _KG)], ib)
            ca = pltpu.async_copy(ta.at[ia], ba, sa)
            cb = pltpu.async_copy(tb.at[ib], bb, sb)
            ca.wait()
            cb.wait()
            pltpu.sync_copy(ba, out_a.at[pl.ds(off, _KG)])
            pltpu.sync_copy(bb, out_b.at[pl.ds(off, _KG)])
            return carry

        lax.fori_loop(0, chunks, body, 0)

    return gk(table_a, table_b, dst, src)


def _sc_gather_x(x4flat, dst, src):
    e = dst.shape[0]
    n4 = x4flat.shape[0]
    per_w = e // _NW
    chunks = per_w // _KX
    groups = _KX // 16
    mesh = plsc.VectorSubcoreMesh(core_axis_name="c", subcore_axis_name="s")

    @functools.partial(
        pl.kernel, mesh=mesh, compiler_params=_SC_PARAMS,
        out_type=jax.ShapeDtypeStruct((e * 16,), F32),
        scratch_types=[
            pltpu.VMEM((n4,), F32),
            pltpu.VMEM((_KX,), jnp.int32),
            pltpu.VMEM((_KX,), jnp.int32),
            pltpu.VMEM((_KX * 16,), F32),
        ])
    def gxk(x_h, d_h, s_h, out, xt, ia, ib, gx):
        wid = lax.axis_index("s") * _NC + lax.axis_index("c")
        base = wid * per_w
        pltpu.sync_copy(x_h, xt)
        iota = lax.iota(jnp.int32, 16)

        def body(k, carry):
            off = base + k * _KX
            pltpu.sync_copy(d_h.at[pl.ds(off, _KX)], ia)
            pltpu.sync_copy(s_h.at[pl.ds(off, _KX)], ib)

            def grp(j, carry2):
                dv = ia[pl.ds(j * 16, 16)]
                sv = ib[pl.ds(j * 16, 16)]
                flat0 = (j * 16 + iota) * 16
                for c in range(3):
                    xd = plsc.load_gather(xt, [dv * 4 + c])
                    xs = plsc.load_gather(xt, [sv * 4 + c])
                    plsc.store_scatter(gx, [flat0 + c], xd - xs)
                return carry2

            lax.fori_loop(0, groups, grp, 0)
            pltpu.sync_copy(gx, out.at[pl.ds(off * 16, _KX * 16)])
            return carry

        lax.fori_loop(0, chunks, body, 0)

    return gxk(x4flat, dst, src)


def _sc_scatter_h(om, dst, zeros_h):
    e = om.shape[0]
    kg = 200
    per_w = e // _NW
    chunks = per_w // kg
    rows_t = _NPAD // _NS
    mesh = plsc.VectorSubcoreMesh(core_axis_name="c", subcore_axis_name="s")

    @functools.partial(
        pl.kernel, mesh=mesh, compiler_params=_SC_PARAMS,
        out_type=jax.ShapeDtypeStruct((2 * _NPAD, _H), F32),
        scratch_types=[
            pltpu.VMEM((kg,), jnp.int32),
            pltpu.VMEM((kg, _H), F32),
            pltpu.VMEM_SHARED((_NPAD, _H), F32),
        ])
    def sk(m_h, d_h, z_h, out_h, idx, buf, acc):
        c_ax = lax.axis_index("c")
        s_ax = lax.axis_index("s")
        wid = s_ax * _NC + c_ax
        r0 = s_ax * rows_t
        pltpu.sync_copy(z_h.at[pl.ds(r0, rows_t)], acc.at[pl.ds(r0, rows_t)])
        plsc.subcore_barrier()
        base = wid * per_w

        def body(k, carry):
            off = base + k * kg
            pltpu.sync_copy(d_h.at[pl.ds(off, kg)], idx)
            pltpu.sync_copy(m_h.at[pl.ds(off, kg)], buf)
            pltpu.sync_copy(buf, acc.at[idx], add=True)
            return carry

        lax.fori_loop(0, chunks, body, 0)
        plsc.subcore_barrier()
        pltpu.sync_copy(acc.at[pl.ds(r0, rows_t)],
                        out_h.at[pl.ds(c_ax * _NPAD + r0, rows_t)])

    return sk(om, dst, zeros_h)


def _sc_scatter_x(oxflat, dst, zeros_x):
    e = dst.shape[0]
    per_w = e // _NW
    chunks = per_w // _KG
    groups = _KG // 16
    mesh = plsc.VectorSubcoreMesh(core_axis_name="c", subcore_axis_name="s")

    @functools.partial(
        pl.kernel, mesh=mesh, compiler_params=_SC_PARAMS,
        out_type=jax.ShapeDtypeStruct((_NW, _NPAD * 4), F32),
        scratch_types=[
            pltpu.VMEM((_KG,), jnp.int32),
            pltpu.VMEM((_KG * 16,), F32),
            pltpu.VMEM((_NPAD * 4,), F32),
        ])
    def sxk(x_h, d_h, zx_h, out_x, idx, bufx, dxacc):
        wid = lax.axis_index("s") * _NC + lax.axis_index("c")
        pltpu.sync_copy(zx_h, dxacc)
        base = wid * per_w
        iota = lax.iota(jnp.int32, 16)

        def body(k, carry):
            off = base + k * _KG
            pltpu.sync_copy(d_h.at[pl.ds(off, _KG)], idx)
            pltpu.sync_copy(x_h.at[pl.ds(off * 16, _KG * 16)], bufx)

            def grp(j, carry2):
                dv = idx[pl.ds(j * 16, 16)]
                flat0 = (j * 16 + iota) * 16
                for c in range(3):
                    val = plsc.load_gather(bufx, [flat0 + c])
                    plsc.addupdate_scatter(dxacc, [dv * 4 + c], val)
                return carry2

            lax.fori_loop(0, groups, grp, 0)
            return carry

        lax.fori_loop(0, chunks, body, 0)
        pltpu.sync_copy(dxacc, out_x.at[wid])

    return sxk(oxflat, dst, zeros_x)


def _prep_tables(h, w0a, w0b, b0):
    n = h.shape[0]

    def body(h_ref, wa_ref, wb_ref, b0_ref, a_ref, b_ref):
        hv = h_ref[...]
        a_ref[...] = jnp.dot(hv, wa_ref[...], preferred_element_type=F32) + b0_ref[...]
        b_ref[...] = jnp.dot(hv, wb_ref[...], preferred_element_type=F32)

    return pl.pallas_call(
        body,
        grid=(n // _RB,),
        in_specs=[
            pl.BlockSpec((_RB, _H), lambda i: (i, 0)),
            pl.BlockSpec((_H, _H), lambda i: (0, 0)),
            pl.BlockSpec((_H, _H), lambda i: (0, 0)),
            pl.BlockSpec((1, _H), lambda i: (0, 0)),
        ],
        out_specs=[pl.BlockSpec((_RB, _H), lambda i: (i, 0)),
                   pl.BlockSpec((_RB, _H), lambda i: (i, 0))],
        out_shape=[jax.ShapeDtypeStruct((n, _H), F32)] * 2,
    )(h, w0a, w0b, b0)


def _edge_mlp(g_a, g_b, g_x, w0c, w1, b1, infw, infb, xw0, xb0, xw1r, offs, msk):
    e = g_a.shape[0]

    def body(ga_ref, gb_ref, gx_ref, w0c_ref, w1_ref, b1_ref, iw_ref, ib_ref,
             xw0_ref, xb0_ref, xw1_ref, off_ref, msk_ref, om_ref, ox_ref):
        gsum = ga_ref[...] + gb_ref[...]
        relx = jnp.where(msk_ref[...] > 0.5, gx_ref[...][:, :8], 0.0)
        dsq = jnp.sum(relx * relx, axis=1, keepdims=True)
        dist = jnp.sqrt(dsq + 1e-8)
        dfeat = jnp.exp(_COEFF * (dist - off_ref[...]) ** 2)
        t = gsum + jnp.dot(dfeat, w0c_ref[...], preferred_element_type=F32)
        mij = _silu(t)
        m2 = _silu(jnp.dot(mij, w1_ref[...], preferred_element_type=F32)
                   + b1_ref[...])
        ei = jax.nn.sigmoid(jnp.sum(m2 * iw_ref[...], axis=1, keepdims=True)
                            + ib_ref[0, 0])
        om_ref[...] = m2 * ei
        h1 = _silu(jnp.dot(m2, xw0_ref[...], preferred_element_type=F32)
                   + xb0_ref[...])
        xw = jnp.tanh(jnp.sum(h1 * xw1_ref[...], axis=1, keepdims=True))
        ox = relx * (xw / (dist + 1.0))
        ox_ref[...] = jnp.concatenate([ox, jnp.zeros((_EB, 8), F32)], axis=1)

    wfull = lambda shape: pl.BlockSpec(shape, lambda i: (0, 0))
    return pl.pallas_call(
        body,
        grid=(e // _EB,),
        in_specs=[
            pl.BlockSpec((_EB, _H), lambda i: (i, 0)),
            pl.BlockSpec((_EB, _H), lambda i: (i, 0)),
            pl.BlockSpec((_EB, 16), lambda i: (i, 0)),
            wfull((20, _H)),
            wfull((_H, _H)),
            wfull((1, _H)),
            wfull((1, _H)),
            wfull((1, _H)),
            wfull((_H, _H)),
            wfull((1, _H)),
            wfull((1, _H)),
            wfull((1, 20)),
            wfull((1, 8)),
        ],
        out_specs=[pl.BlockSpec((_EB, _H), lambda i: (i, 0)),
                   pl.BlockSpec((_EB, 16), lambda i: (i, 0))],
        out_shape=[jax.ShapeDtypeStruct((e, _H), F32),
                   jax.ShapeDtypeStruct((e, 16), F32)],
    )(g_a, g_b, g_x, w0c, w1, b1, infw, infb, xw0, xb0, xw1r, offs, msk)


def _x_update(dxp, x4flat):
    n4 = x4flat.shape[0]
    m4 = dxp.shape[1]

    def body(dx_ref, x_ref, x_out):
        dx = jnp.sum(dx_ref[...], axis=0)
        x_out[...] = x_ref[...] + dx[:n4]

    return pl.pallas_call(
        body,
        grid=(1,),
        in_specs=[pl.BlockSpec((_NW, m4), lambda i: (0, 0)),
                  pl.BlockSpec((n4,), lambda i: (0,))],
        out_specs=pl.BlockSpec((n4,), lambda i: (0,)),
        out_shape=jax.ShapeDtypeStruct((n4,), F32),
    )(dxp, x4flat)


def _node_update(p, h, nw0a, nw0b, nb0, nw1, nb1, nxt):
    n = h.shape[0]
    has_next = nxt is not None
    p1_off = _NPAD // _RB

    def body(*refs):
        if has_next:
            (p0_ref, p1_ref, h_ref, wa_ref, wb_ref, b0_ref,
             w1_ref, b1_ref, na_ref, nb_ref, nbias_ref,
             h_out, a_out, b_out) = refs
        else:
            (p0_ref, p1_ref, h_ref, wa_ref, wb_ref, b0_ref,
             w1_ref, b1_ref, h_out) = refs
        mi = p0_ref[...] + p1_ref[...]
        hv = h_ref[...]
        nh = _silu(jnp.dot(mi, wa_ref[...], preferred_element_type=F32)
                   + jnp.dot(hv, wb_ref[...], preferred_element_type=F32)
                   + b0_ref[...])
        hn = hv + jnp.dot(nh, w1_ref[...], preferred_element_type=F32) + b1_ref[...]
        h_out[...] = hn
        if has_next:
            a_out[...] = jnp.dot(hn, na_ref[...], preferred_element_type=F32) + nbias_ref[...]
            b_out[...] = jnp.dot(hn, nb_ref[...], preferred_element_type=F32)

    wfull = lambda shape: pl.BlockSpec(shape, lambda i: (0, 0))
    in_specs = [
        pl.BlockSpec((_RB, _H), lambda i: (i, 0)),
        pl.BlockSpec((_RB, _H), lambda i: (i + p1_off, 0)),
        pl.BlockSpec((_RB, _H), lambda i: (i, 0)),
        wfull((_H, _H)),
        wfull((_H, _H)),
        wfull((1, _H)),
        wfull((_H, _H)),
        wfull((1, _H)),
    ]
    out_specs = [pl.BlockSpec((_RB, _H), lambda i: (i, 0))]
    out_shape = [jax.ShapeDtypeStruct((n, _H), F32)]
    args = [p, p, h, nw0a, nw0b, nb0, nw1, nb1]
    if has_next:
        in_specs += [wfull((_H, _H)), wfull((_H, _H)), wfull((1, _H))]
        out_specs += [pl.BlockSpec((_RB, _H), lambda i: (i, 0)),
                      pl.BlockSpec((_RB, _H), lambda i: (i, 0))]
        out_shape += [jax.ShapeDtypeStruct((n, _H), F32)] * 2
        args += list(nxt)
    res = pl.pallas_call(
        body,
        grid=(n // _RB,),
        in_specs=in_specs,
        out_specs=out_specs,
        out_shape=out_shape,
    )(*args)
    return res


def kernel(h, x, edge_index, mask_ligand, edge_w0, edge_b0, edge_w1, edge_b1,
           inf_w, inf_b, x_w0, x_b0, x_w1, node_w0, node_b0, node_w1, node_b1):
    del mask_ligand
    src = edge_index[0].astype(jnp.int32)
    dst = edge_index[1].astype(jnp.int32)
    e = src.shape[0]
    x4flat = jnp.pad(x.astype(F32), ((0, 0), (0, 1))).reshape(-1)
    zeros_h = jnp.zeros((_NPAD, _H), F32)
    zeros_x = jnp.zeros((_NPAD * 4,), F32)
    offs = jnp.asarray(_OFFSET, F32).reshape(1, 20)
    msk = jnp.asarray([1., 1., 1., 0., 0., 0., 0., 0.], F32).reshape(1, 8)

    tab = lambda l: (edge_w0[l, :128], edge_w0[l, 128:256],
                     edge_b0[l].reshape(1, 128))
    a_t, b_t = _prep_tables(h, *tab(0))
    for l in range(2):
        g_a, g_b = _sc_gather(a_t, b_t, dst, src)
        g_x = _sc_gather_x(x4flat, dst, src).reshape(e, 16)
        om, ox = _edge_mlp(
            g_a, g_b, g_x, edge_w0[l, 256:276], edge_w1[l],
            edge_b1[l].reshape(1, 128), inf_w[l].reshape(1, 128),
            jnp.broadcast_to(inf_b[l].reshape(1, 1), (1, 128)),
            x_w0[l], x_b0[l].reshape(1, 128), x_w1[l].reshape(1, 128),
            offs, msk)
        p = _sc_scatter_h(om, dst, zeros_h)
        dxp = _sc_scatter_x(ox.reshape(-1), dst, zeros_x)
        nxt = tab(1) if l == 0 else None
        res = _node_update(p, h, node_w0[l, :128], node_w0[l, 128:256],
                           node_b0[l].reshape(1, 128), node_w1[l],
                           node_b1[l].reshape(1, 128), nxt)
        x4flat = _x_update(dxp, x4flat)
        if l == 0:
            h, a_t, b_t = res
        else:
            (h,) = res
    return h, x4flat.reshape(-1, 4)[:, :3]
